# baseline (speedup 1.0000x reference)
import functools

import jax
import jax.numpy as jnp
from jax import lax
from jax.experimental import pallas as pl
from jax.experimental.pallas import tpu as pltpu
from jax.experimental.pallas import tpu_sc as plsc

_C0 = 0.28209479177387814
_VOXEL = 64
_FAR = 100.0

_NS = 16
_NC = 2
_NW = _NC * _NS


def _sc_mask_kernel(n_pts, n_pcd):
    table_n = _VOXEL * _VOXEL * _VOXEL
    ppt = n_pts // _NW
    ppc = n_pcd // _NS
    assert ppt == ppc, "phases share index/key buffer shapes"
    rows = ppt // 16
    zslab = table_n // _NS

    mesh = plsc.VectorSubcoreMesh(core_axis_name="core", subcore_axis_name="subcore")

    @functools.partial(
        pl.kernel,
        mesh=mesh,
        out_type=jax.ShapeDtypeStruct((_NW, ppt), jnp.float32),
        scratch_types=[
            pltpu.VMEM_SHARED((table_n,), jnp.float32),
            pltpu.VMEM((3 * ppt,), jnp.int32),
            pltpu.VMEM((3 * ppt,), jnp.int32),
            pltpu.VMEM((3 * ppt,), jnp.int32),
            pltpu.VMEM((3 * ppt,), jnp.int32),
            pltpu.VMEM((ppt,), jnp.int32),
            pltpu.VMEM((ppt,), jnp.float32),
            pltpu.SemaphoreType.DMA,
            pltpu.SemaphoreType.DMA,
            pltpu.SemaphoreType.DMA,
        ],
    )
    def mask_kernel(cwords, pwords, zeros_hbm, mask_out,
                    table, idx1, idx2, dsl1, dsl2, keybuf, valbuf,
                    sem0, sem1, sem2):
        i32 = jnp.int32
        sid = lax.axis_index("subcore").astype(i32)
        wid = lax.axis_index("core").astype(i32) * i32(_NS) + sid
        iota6 = lax.iota(i32, 16) * i32(6)
        ones16 = jnp.full((16,), 1.0, jnp.float32)

        def build_idx(idx, base_pt):
            for p in range(3):
                @pl.loop(0, rows)
                def _r(r, _p=p):
                    b = i32(6) * base_pt + i32(96) * r + i32(2 * _p)
                    idx[pl.ds(_p * ppt + 16 * r, 16)] = b + iota6

        def compute_keys(dsl):
            @pl.loop(0, rows)
            def _r(r):
                o = 16 * r
                keybuf[pl.ds(o, 16)] = (
                    dsl[pl.ds(o, 16)]
                    + dsl[pl.ds(ppt + o, 16)] * i32(_VOXEL)
                    + dsl[pl.ds(2 * ppt + o, 16)] * i32(_VOXEL * _VOXEL))

        build_idx(idx2, wid * i32(ppt))
        g2 = pltpu.async_copy(cwords.at[idx2], dsl2, sem2)
        build_idx(idx1, sid * i32(ppc))
        g1 = pltpu.async_copy(pwords.at[idx1], dsl1, sem1)
        z0 = pltpu.async_copy(zeros_hbm, table.at[pl.ds(sid * i32(zslab), zslab)],
                              sem0)

        @pl.loop(0, rows)
        def _ones(r):
            valbuf[pl.ds(16 * r, 16)] = ones16

        g1.wait()
        compute_keys(dsl1)
        z0.wait()
        plsc.subcore_barrier()
        pltpu.sync_copy(valbuf, table.at[keybuf])
        plsc.subcore_barrier()

        g2.wait()
        compute_keys(dsl2)
        pltpu.sync_copy(table.at[keybuf], valbuf)
        pltpu.sync_copy(valbuf, mask_out.at[wid])

    return mask_kernel


def _stats_body(g_ref, mean_ref, scale_ref):
    x = g_ref[...]
    cnt = x.size
    s = jnp.sum(x)
    ss = jnp.sum(x * x)
    mean = s / cnt
    var = (ss - cnt * mean * mean) / (cnt - 1)
    mean_ref[0, 0] = mean
    scale_ref[0, 0] = (2.0 * _FAR / _VOXEL / 6.0) / jnp.sqrt(var)


def _map_body(mean_ref, scale_ref, g_ref, m_ref, o_ref):
    g = g_ref[...]
    mean = mean_ref[0, 0]
    dmscale = scale_ref[0, 0]

    dm = (g[0:3] - mean) * dmscale
    quat = g[3:7]
    sg = jax.nn.sigmoid(g[7:10])
    scale = sg * 2.0 * _FAR / _VOXEL
    opa = jax.nn.sigmoid(g[10:11] - 4.0)
    d1 = (jax.nn.sigmoid(g[11:14]) - 0.5) / _C0
    d2 = g[14:23] / 20.0
    d3 = g[23:38] / 40.0
    d4 = g[38:59] / 80.0

    qn = quat / jnp.sqrt(jnp.sum(quat * quat, axis=0, keepdims=True))
    r_, x_, y_, z_ = qn[0:1], qn[1:2], qn[2:3], qn[3:4]
    r00 = 1.0 - 2.0 * (y_ * y_ + z_ * z_)
    r01 = 2.0 * (x_ * y_ - r_ * z_)
    r02 = 2.0 * (x_ * z_ + r_ * y_)
    r10 = 2.0 * (x_ * y_ + r_ * z_)
    r11 = 1.0 - 2.0 * (x_ * x_ + z_ * z_)
    r12 = 2.0 * (y_ * z_ - r_ * x_)
    r20 = 2.0 * (x_ * z_ - r_ * y_)
    r21 = 2.0 * (y_ * z_ + r_ * x_)
    r22 = 1.0 - 2.0 * (x_ * x_ + y_ * y_)
    s0, s1, s2 = scale[0:1], scale[1:2], scale[2:3]
    l00, l01, l02 = r00 * s0, r01 * s1, r02 * s2
    l10, l11, l12 = r10 * s0, r11 * s1, r12 * s2
    l20, l21, l22 = r20 * s0, r21 * s1, r22 * s2
    c00 = l00 * l00 + l01 * l01 + l02 * l02
    c01 = l00 * l10 + l01 * l11 + l02 * l12
    c02 = l00 * l20 + l01 * l21 + l02 * l22
    c11 = l10 * l10 + l11 * l11 + l12 * l12
    c12 = l10 * l20 + l11 * l21 + l12 * l22
    c22 = l20 * l20 + l21 * l21 + l22 * l22

    maskrow = (m_ref[...] > 0.0).astype(jnp.float32)

    o_ref[...] = jnp.concatenate(
        [dm, quat, scale, opa, d1, d2, d3, d4,
         c00, c01, c02, c01, c11, c12, c02, c12, c22, maskrow], axis=0)


_MAP_CH = 32


def kernel(gaussian_features, coordinates, pcd_coords):
    cwords = lax.bitcast_convert_type(coordinates, jnp.int32).reshape(-1)
    pwords = lax.bitcast_convert_type(pcd_coords, jnp.int32).reshape(-1)
    with jax.enable_x64(False):
        return _kernel_x32(gaussian_features, cwords, pwords,
                           pcd_coords.shape[0])


def _kernel_x32(gf, cwords, pwords, m):
    n = gf.shape[1]
    nb = n // 128

    table_n = _VOXEL * _VOXEL * _VOXEL
    zeros_slab = jnp.zeros((table_n // _NS,), jnp.float32)

    mask3d = _sc_mask_kernel(n, m)(cwords, pwords, zeros_slab)
    mask = mask3d.reshape(1, nb, 128)

    gf3d = gf.reshape(59, nb, 128)
    mean, dmscale = pl.pallas_call(
        _stats_body,
        grid=(1,),
        out_shape=[jax.ShapeDtypeStruct((1, 1), jnp.float32)] * 2,
        in_specs=[pl.BlockSpec((3, nb, 128), lambda i: (0, 0, 0))],
        out_specs=[pl.BlockSpec((1, 1), lambda i: (0, 0),
                                memory_space=pltpu.SMEM)] * 2,
    )(gf3d)

    out3 = pl.pallas_call(
        _map_body,
        grid=(nb // _MAP_CH,),
        in_specs=[
            pl.BlockSpec((1, 1), lambda i: (0, 0), memory_space=pltpu.SMEM),
            pl.BlockSpec((1, 1), lambda i: (0, 0), memory_space=pltpu.SMEM),
            pl.BlockSpec((59, _MAP_CH, 128), lambda i: (0, i, 0)),
            pl.BlockSpec((1, _MAP_CH, 128), lambda i: (0, i, 0)),
        ],
        out_specs=pl.BlockSpec((69, _MAP_CH, 128), lambda i: (0, i, 0)),
        out_shape=jax.ShapeDtypeStruct((69, nb, 128), jnp.float32),
    )(mean, dmscale, gf3d, mask)
    return out3.reshape(69, n)


# speedup vs baseline: 1.0333x; 1.0333x over previous
import functools

import jax
import jax.numpy as jnp
from jax import lax
from jax.experimental import pallas as pl
from jax.experimental.pallas import tpu as pltpu
from jax.experimental.pallas import tpu_sc as plsc

_C0 = 0.28209479177387814
_VOXEL = 64
_FAR = 100.0

_NS = 16
_NC = 2
_NW = _NC * _NS


def _sc_mask_kernel(n_pts, n_pcd):
    table_n = _VOXEL * _VOXEL * _VOXEL
    ppt = n_pts // _NW
    ppc = n_pcd // _NS
    assert ppt == ppc, "phases share index/key buffer shapes"
    rows = ppt // 16
    wpt = 6 * ppt
    zslab = table_n // _NS

    mesh = plsc.VectorSubcoreMesh(core_axis_name="core", subcore_axis_name="subcore")

    @functools.partial(
        pl.kernel,
        mesh=mesh,
        out_type=jax.ShapeDtypeStruct((_NW, ppt), jnp.float32),
        scratch_types=[
            pltpu.VMEM_SHARED((table_n,), jnp.float32),
            pltpu.VMEM_SHARED((_NS * wpt,), jnp.int32),
            pltpu.VMEM((3 * ppt,), jnp.int32),
            pltpu.VMEM((ppt,), jnp.int32),
            pltpu.VMEM((ppt,), jnp.int32),
            pltpu.VMEM((ppt,), jnp.float32),
            pltpu.SemaphoreType.DMA,
            pltpu.SemaphoreType.DMA,
            pltpu.SemaphoreType.DMA,
        ],
    )
    def mask_kernel(cwords, pwords, zeros_hbm, mask_out,
                    table, wslab, dsl, idxbuf, keybuf, valbuf,
                    sem0, sem1, sem2):
        i32 = jnp.int32
        sid = lax.axis_index("subcore").astype(i32)
        wid = lax.axis_index("core").astype(i32) * i32(_NS) + sid
        iota6 = lax.iota(i32, 16) * i32(6)
        ones16 = jnp.full((16,), 1.0, jnp.float32)
        sbase = sid * i32(wpt)

        def build_idx():
            @pl.loop(0, rows)
            def _r(r):
                idxbuf[pl.ds(16 * r, 16)] = sbase + i32(96) * r + iota6

        def shift_idx():
            @pl.loop(0, rows)
            def _r(r):
                s = pl.ds(16 * r, 16)
                idxbuf[s] = idxbuf[s] + i32(2)

        def deinterleave():
            pltpu.sync_copy(wslab.at[idxbuf], dsl.at[pl.ds(0, ppt)])
            shift_idx()
            pltpu.sync_copy(wslab.at[idxbuf], dsl.at[pl.ds(ppt, ppt)])
            shift_idx()
            pltpu.sync_copy(wslab.at[idxbuf], dsl.at[pl.ds(2 * ppt, ppt)])

        def compute_keys():
            @pl.loop(0, rows)
            def _r(r):
                o = 16 * r
                keybuf[pl.ds(o, 16)] = (
                    dsl[pl.ds(o, 16)]
                    + dsl[pl.ds(ppt + o, 16)] * i32(_VOXEL)
                    + dsl[pl.ds(2 * ppt + o, 16)] * i32(_VOXEL * _VOXEL))

        s1 = pltpu.async_copy(pwords.at[pl.ds(sid * i32(wpt), wpt)],
                              wslab.at[pl.ds(sid * i32(wpt), wpt)], sem1)
        z0 = pltpu.async_copy(zeros_hbm, table.at[pl.ds(sid * i32(zslab), zslab)],
                              sem0)
        build_idx()

        @pl.loop(0, rows)
        def _ones(r):
            valbuf[pl.ds(16 * r, 16)] = ones16

        s1.wait()
        deinterleave()
        s2 = pltpu.async_copy(cwords.at[pl.ds(wid * i32(wpt), wpt)],
                              wslab.at[pl.ds(sid * i32(wpt), wpt)], sem2)
        compute_keys()
        z0.wait()
        plsc.subcore_barrier()
        pltpu.sync_copy(valbuf, table.at[keybuf])
        plsc.subcore_barrier()

        s2.wait()
        build_idx()
        deinterleave()
        compute_keys()
        pltpu.sync_copy(table.at[keybuf], valbuf)
        pltpu.sync_copy(valbuf, mask_out.at[wid])

    return mask_kernel


def _stats_body(g_ref, mean_ref, scale_ref):
    x = g_ref[...]
    cnt = x.size
    s = jnp.sum(x)
    ss = jnp.sum(x * x)
    mean = s / cnt
    var = (ss - cnt * mean * mean) / (cnt - 1)
    mean_ref[0, 0] = mean
    scale_ref[0, 0] = (2.0 * _FAR / _VOXEL / 6.0) / jnp.sqrt(var)


def _map_body(mean_ref, scale_ref, g_ref, m_ref, o_ref):
    g = g_ref[...]
    mean = mean_ref[0, 0]
    dmscale = scale_ref[0, 0]

    dm = (g[0:3] - mean) * dmscale
    quat = g[3:7]
    sg = jax.nn.sigmoid(g[7:10])
    scale = sg * 2.0 * _FAR / _VOXEL
    opa = jax.nn.sigmoid(g[10:11] - 4.0)
    d1 = (jax.nn.sigmoid(g[11:14]) - 0.5) / _C0
    d2 = g[14:23] / 20.0
    d3 = g[23:38] / 40.0
    d4 = g[38:59] / 80.0

    qn = quat / jnp.sqrt(jnp.sum(quat * quat, axis=0, keepdims=True))
    r_, x_, y_, z_ = qn[0:1], qn[1:2], qn[2:3], qn[3:4]
    r00 = 1.0 - 2.0 * (y_ * y_ + z_ * z_)
    r01 = 2.0 * (x_ * y_ - r_ * z_)
    r02 = 2.0 * (x_ * z_ + r_ * y_)
    r10 = 2.0 * (x_ * y_ + r_ * z_)
    r11 = 1.0 - 2.0 * (x_ * x_ + z_ * z_)
    r12 = 2.0 * (y_ * z_ - r_ * x_)
    r20 = 2.0 * (x_ * z_ - r_ * y_)
    r21 = 2.0 * (y_ * z_ + r_ * x_)
    r22 = 1.0 - 2.0 * (x_ * x_ + y_ * y_)
    s0, s1, s2 = scale[0:1], scale[1:2], scale[2:3]
    l00, l01, l02 = r00 * s0, r01 * s1, r02 * s2
    l10, l11, l12 = r10 * s0, r11 * s1, r12 * s2
    l20, l21, l22 = r20 * s0, r21 * s1, r22 * s2
    c00 = l00 * l00 + l01 * l01 + l02 * l02
    c01 = l00 * l10 + l01 * l11 + l02 * l12
    c02 = l00 * l20 + l01 * l21 + l02 * l22
    c11 = l10 * l10 + l11 * l11 + l12 * l12
    c12 = l10 * l20 + l11 * l21 + l12 * l22
    c22 = l20 * l20 + l21 * l21 + l22 * l22

    maskrow = (m_ref[...] > 0.0).astype(jnp.float32)

    o_ref[...] = jnp.concatenate(
        [dm, quat, scale, opa, d1, d2, d3, d4,
         c00, c01, c02, c01, c11, c12, c02, c12, c22, maskrow], axis=0)


_MAP_CH = 32


def kernel(gaussian_features, coordinates, pcd_coords):
    cwords = lax.bitcast_convert_type(coordinates, jnp.int32).reshape(-1)
    pwords = lax.bitcast_convert_type(pcd_coords, jnp.int32).reshape(-1)
    with jax.enable_x64(False):
        return _kernel_x32(gaussian_features, cwords, pwords,
                           pcd_coords.shape[0])


def _kernel_x32(gf, cwords, pwords, m):
    n = gf.shape[1]
    nb = n // 128

    table_n = _VOXEL * _VOXEL * _VOXEL
    zeros_slab = jnp.zeros((table_n // _NS,), jnp.float32)

    mask3d = _sc_mask_kernel(n, m)(cwords, pwords, zeros_slab)
    mask = mask3d.reshape(1, nb, 128)

    gf3d = gf.reshape(59, nb, 128)
    mean, dmscale = pl.pallas_call(
        _stats_body,
        grid=(1,),
        out_shape=[jax.ShapeDtypeStruct((1, 1), jnp.float32)] * 2,
        in_specs=[pl.BlockSpec((3, nb, 128), lambda i: (0, 0, 0))],
        out_specs=[pl.BlockSpec((1, 1), lambda i: (0, 0),
                                memory_space=pltpu.SMEM)] * 2,
    )(gf3d)

    out3 = pl.pallas_call(
        _map_body,
        grid=(nb // _MAP_CH,),
        in_specs=[
            pl.BlockSpec((1, 1), lambda i: (0, 0), memory_space=pltpu.SMEM),
            pl.BlockSpec((1, 1), lambda i: (0, 0), memory_space=pltpu.SMEM),
            pl.BlockSpec((59, _MAP_CH, 128), lambda i: (0, i, 0)),
            pl.BlockSpec((1, _MAP_CH, 128), lambda i: (0, i, 0)),
        ],
        out_specs=pl.BlockSpec((69, _MAP_CH, 128), lambda i: (0, i, 0)),
        out_shape=jax.ShapeDtypeStruct((69, nb, 128), jnp.float32),
    )(mean, dmscale, gf3d, mask)
    return out3.reshape(69, n)


# speedup vs baseline: 5.5518x; 5.3729x over previous
import functools

import jax
import jax.numpy as jnp
from jax import lax
from jax.experimental import pallas as pl
from jax.experimental.pallas import tpu as pltpu
from jax.experimental.pallas import tpu_sc as plsc

_C0 = 0.28209479177387814
_VOXEL = 64
_FAR = 100.0

_NS = 16
_NC = 2
_NW = _NC * _NS


def _sc_mask_kernel(n_pts, n_pcd):
    table_n = _VOXEL * _VOXEL * _VOXEL
    ppt = n_pts // _NW
    ppc = n_pcd // _NS
    assert ppt == ppc, "phases share index/key buffer shapes"
    rows = ppt // 16
    zslab = table_n // _NS

    mesh = plsc.VectorSubcoreMesh(core_axis_name="core", subcore_axis_name="subcore")

    @functools.partial(
        pl.kernel,
        mesh=mesh,
        out_type=jax.ShapeDtypeStruct((_NW, ppt), jnp.float32),
        scratch_types=[
            pltpu.VMEM_SHARED((table_n,), jnp.float32),
            pltpu.VMEM((ppt,), jnp.int32),
            pltpu.VMEM((ppt,), jnp.int32),
            pltpu.VMEM((ppt,), jnp.int32),
            pltpu.VMEM((ppt,), jnp.int32),
            pltpu.VMEM((ppt,), jnp.float32),
            pltpu.SemaphoreType.DMA,
            pltpu.SemaphoreType.DMA,
        ],
    )
    def mask_kernel(cx, cy, cz, px, py, pz, zeros_hbm, mask_out,
                    table, xb, yb, zb, keybuf, valbuf, sem0, sem1):
        i32 = jnp.int32
        sid = lax.axis_index("subcore").astype(i32)
        wid = lax.axis_index("core").astype(i32) * i32(_NS) + sid
        ones16 = jnp.full((16,), 1.0, jnp.float32)

        def stage(sx, sy, sz, base, count, sem):
            c0 = pltpu.async_copy(sx.at[pl.ds(base, count)], xb, sem)
            c1 = pltpu.async_copy(sy.at[pl.ds(base, count)], yb, sem)
            c2 = pltpu.async_copy(sz.at[pl.ds(base, count)], zb, sem)
            return c0, c1, c2

        def compute_keys():
            @pl.loop(0, rows)
            def _r(r):
                s = pl.ds(16 * r, 16)
                keybuf[s] = (xb[s]
                             + yb[s] * i32(_VOXEL)
                             + zb[s] * i32(_VOXEL * _VOXEL))

        cps = stage(px, py, pz, sid * i32(ppc), ppc, sem1)
        z0 = pltpu.async_copy(zeros_hbm, table.at[pl.ds(sid * i32(zslab), zslab)],
                              sem0)

        @pl.loop(0, rows)
        def _ones(r):
            valbuf[pl.ds(16 * r, 16)] = ones16

        for c in cps:
            c.wait()
        compute_keys()
        z0.wait()
        plsc.subcore_barrier()
        pltpu.sync_copy(valbuf, table.at[keybuf])
        plsc.subcore_barrier()

        cps = stage(cx, cy, cz, wid * i32(ppt), ppt, sem1)
        for c in cps:
            c.wait()
        compute_keys()
        pltpu.sync_copy(table.at[keybuf], valbuf)
        pltpu.sync_copy(valbuf, mask_out.at[wid])

    return mask_kernel


def _stats_body(g_ref, mean_ref, scale_ref):
    x = g_ref[...]
    cnt = x.size
    s = jnp.sum(x)
    ss = jnp.sum(x * x)
    mean = s / cnt
    var = (ss - cnt * mean * mean) / (cnt - 1)
    mean_ref[0, 0] = mean
    scale_ref[0, 0] = (2.0 * _FAR / _VOXEL / 6.0) / jnp.sqrt(var)


def _map_body(mean_ref, scale_ref, g_ref, m_ref, o_ref):
    g = g_ref[...]
    mean = mean_ref[0, 0]
    dmscale = scale_ref[0, 0]

    dm = (g[0:3] - mean) * dmscale
    quat = g[3:7]
    sg = jax.nn.sigmoid(g[7:10])
    scale = sg * 2.0 * _FAR / _VOXEL
    opa = jax.nn.sigmoid(g[10:11] - 4.0)
    d1 = (jax.nn.sigmoid(g[11:14]) - 0.5) / _C0
    d2 = g[14:23] / 20.0
    d3 = g[23:38] / 40.0
    d4 = g[38:59] / 80.0

    qn = quat / jnp.sqrt(jnp.sum(quat * quat, axis=0, keepdims=True))
    r_, x_, y_, z_ = qn[0:1], qn[1:2], qn[2:3], qn[3:4]
    r00 = 1.0 - 2.0 * (y_ * y_ + z_ * z_)
    r01 = 2.0 * (x_ * y_ - r_ * z_)
    r02 = 2.0 * (x_ * z_ + r_ * y_)
    r10 = 2.0 * (x_ * y_ + r_ * z_)
    r11 = 1.0 - 2.0 * (x_ * x_ + z_ * z_)
    r12 = 2.0 * (y_ * z_ - r_ * x_)
    r20 = 2.0 * (x_ * z_ - r_ * y_)
    r21 = 2.0 * (y_ * z_ + r_ * x_)
    r22 = 1.0 - 2.0 * (x_ * x_ + y_ * y_)
    s0, s1, s2 = scale[0:1], scale[1:2], scale[2:3]
    l00, l01, l02 = r00 * s0, r01 * s1, r02 * s2
    l10, l11, l12 = r10 * s0, r11 * s1, r12 * s2
    l20, l21, l22 = r20 * s0, r21 * s1, r22 * s2
    c00 = l00 * l00 + l01 * l01 + l02 * l02
    c01 = l00 * l10 + l01 * l11 + l02 * l12
    c02 = l00 * l20 + l01 * l21 + l02 * l22
    c11 = l10 * l10 + l11 * l11 + l12 * l12
    c12 = l10 * l20 + l11 * l21 + l12 * l22
    c22 = l20 * l20 + l21 * l21 + l22 * l22

    maskrow = (m_ref[...] > 0.0).astype(jnp.float32)

    o_ref[...] = jnp.concatenate(
        [dm, quat, scale, opa, d1, d2, d3, d4,
         c00, c01, c02, c01, c11, c12, c02, c12, c22, maskrow], axis=0)


_MAP_CH = 32


def kernel(gaussian_features, coordinates, pcd_coords):
    with jax.enable_x64(False):
        c32 = coordinates.astype(jnp.int32)
        p32 = pcd_coords.astype(jnp.int32)
        return _kernel_x32(gaussian_features, c32, p32)


def _kernel_x32(gf, c32, p32):
    n = c32.shape[0]
    m = p32.shape[0]
    nb = n // 128

    table_n = _VOXEL * _VOXEL * _VOXEL
    zeros_slab = jnp.zeros((table_n // _NS,), jnp.float32)

    cx, cy, cz = (c32[:, 0], c32[:, 1], c32[:, 2])
    px, py, pz = (p32[:, 0], p32[:, 1], p32[:, 2])
    mask3d = _sc_mask_kernel(n, m)(cx, cy, cz, px, py, pz, zeros_slab)
    mask = mask3d.reshape(1, nb, 128)

    gf3d = gf.reshape(59, nb, 128)
    mean, dmscale = pl.pallas_call(
        _stats_body,
        grid=(1,),
        out_shape=[jax.ShapeDtypeStruct((1, 1), jnp.float32)] * 2,
        in_specs=[pl.BlockSpec((3, nb, 128), lambda i: (0, 0, 0))],
        out_specs=[pl.BlockSpec((1, 1), lambda i: (0, 0),
                                memory_space=pltpu.SMEM)] * 2,
    )(gf3d)

    out3 = pl.pallas_call(
        _map_body,
        grid=(nb // _MAP_CH,),
        in_specs=[
            pl.BlockSpec((1, 1), lambda i: (0, 0), memory_space=pltpu.SMEM),
            pl.BlockSpec((1, 1), lambda i: (0, 0), memory_space=pltpu.SMEM),
            pl.BlockSpec((59, _MAP_CH, 128), lambda i: (0, i, 0)),
            pl.BlockSpec((1, _MAP_CH, 128), lambda i: (0, i, 0)),
        ],
        out_specs=pl.BlockSpec((69, _MAP_CH, 128), lambda i: (0, i, 0)),
        out_shape=jax.ShapeDtypeStruct((69, nb, 128), jnp.float32),
    )(mean, dmscale, gf3d, mask)
    return out3.reshape(69, n)
